# Initial kernel scaffold; baseline (speedup 1.0000x reference)
import jax
import jax.numpy as jnp
from jax.experimental import pallas as pl


def kernel(x, edge_index, batch_vec, W_gcn, b_gcn, W1, b1, W2, b2):
    raise NotImplementedError("write your pallas kernel here")


# speedup vs baseline: 12.7586x; 12.7586x over previous
import jax
import jax.numpy as jnp
from jax import lax
from jax.experimental import pallas as pl
from jax.experimental.pallas import tpu as pltpu
from jax.experimental.pallas import tpu_sc as plsc

N = 10000
E = 320000
D = 128
B = 64

NC, NS = 2, 16
NW = NC * NS
CH = 128
CPT = 80
E_PAD = NW * CPT * CH
N_PAD = 10240
PAD_ROW = N_PAD - 1
RPT = N_PAD // NS
RB = 1024
GN = N_PAD // RB

_mesh = plsc.VectorSubcoreMesh(
    core_axis_name="c", subcore_axis_name="s", num_cores=NC, num_subcores=NS)
_sc_params = pltpu.CompilerParams(needs_layout_passes=False)


def _deg_body(dst_hbm, out_hbm, dst_v, deg_v):
    c = lax.axis_index("c")
    s = lax.axis_index("s")
    wid = s * NC + c
    pltpu.sync_copy(dst_hbm.at[wid], dst_v)

    def zero(i, carry):
        deg_v[pl.ds(i * 16, 16)] = jnp.zeros((16,), jnp.float32)
        return carry
    lax.fori_loop(0, N_PAD // 16, zero, 0)

    ones16 = jnp.ones((16,), jnp.float32)

    def edge(i, carry):
        idx = dst_v[i]
        plsc.addupdate_scatter(deg_v, [idx], ones16)
        return carry
    lax.fori_loop(0, (CPT * CH) // 16, edge, 0)

    pltpu.sync_copy(deg_v, out_hbm.at[wid])


_deg_call = pl.kernel(
    _deg_body,
    out_type=jax.ShapeDtypeStruct((NW, N_PAD), jnp.float32),
    mesh=_mesh,
    scratch_types=[
        pltpu.VMEM((N_PAD // 16, 16), jnp.int32),
        pltpu.VMEM((N_PAD,), jnp.float32),
    ],
    compiler_params=_sc_params,
)


def _scale_body(x_ref, degp_ref, W_ref, q_ref, dinv_ref):
    deg = jnp.sum(degp_ref[...], axis=0) + 1.0
    dinv = lax.rsqrt(deg)
    dinv_ref[...] = dinv
    h = jnp.dot(x_ref[...], W_ref[...], preferred_element_type=jnp.float32)
    dT = jnp.transpose(dinv)
    for i in range(8):
        q_ref[i * 128:(i + 1) * 128, :] = (
            h[i * 128:(i + 1) * 128, :] * dT[:, i:i + 1])


_scale_call = pl.pallas_call(
    _scale_body,
    grid=(GN,),
    in_specs=[
        pl.BlockSpec((RB, D), lambda g: (g, 0)),
        pl.BlockSpec((NW, RB // 128, 128), lambda g: (0, g, 0)),
        pl.BlockSpec((D, D), lambda g: (0, 0)),
    ],
    out_specs=[
        pl.BlockSpec((RB, D), lambda g: (g, 0)),
        pl.BlockSpec((RB // 128, 128), lambda g: (g, 0)),
    ],
    out_shape=[
        jax.ShapeDtypeStruct((N_PAD, D), jnp.float32),
        jax.ShapeDtypeStruct((N_PAD // 128, 128), jnp.float32),
    ],
)


def _agg_body(y_hbm, src_hbm, dst_hbm, out_hbm, src_v, dst_v, rows, agg_s, sem):
    c = lax.axis_index("c")
    s = lax.axis_index("s")
    wid = s * NC + c
    pltpu.sync_copy(src_hbm.at[wid], src_v)
    pltpu.sync_copy(dst_hbm.at[wid], dst_v)

    def zero(i, carry):
        def lane(k, carry2):
            rows[i, pl.ds(k * 16, 16)] = jnp.zeros((16,), jnp.float32)
            return carry2
        lax.fori_loop(0, D // 16, lane, 0)
        return carry
    lax.fori_loop(0, CH, zero, 0)
    base = s * RPT
    for r in range(RPT // CH):
        pltpu.sync_copy(rows, agg_s.at[pl.ds(base + r * CH, CH)])
    plsc.subcore_barrier()

    def chunk(j, carry):
        pltpu.async_copy(y_hbm.at[src_v.at[j]], rows, sem).wait()
        pltpu.sync_copy(rows, agg_s.at[dst_v.at[j]], add=True)
        return carry
    lax.fori_loop(0, CPT, chunk, 0)

    plsc.subcore_barrier()
    for r in range(RPT // CH):
        pltpu.sync_copy(agg_s.at[pl.ds(base + r * CH, CH)],
                        out_hbm.at[c, pl.ds(base + r * CH, CH)])


_agg_call = pl.kernel(
    _agg_body,
    out_type=jax.ShapeDtypeStruct((NC, N_PAD, D), jnp.float32),
    mesh=_mesh,
    scratch_types=[
        pltpu.VMEM((CPT, CH), jnp.int32),
        pltpu.VMEM((CPT, CH), jnp.int32),
        pltpu.VMEM((CH, D), jnp.float32),
        pltpu.VMEM_SHARED((N_PAD, D), jnp.float32),
        pltpu.SemaphoreType.DMA,
    ],
    compiler_params=_sc_params,
)


def _pool_body(aggp_ref, y_ref, dinv_ref, batch_ref, bg_ref,
               W1_ref, b1_ref, w2p_ref, b2_ref, out_ref, sums_ref, cnt_ref):
    g = pl.program_id(0)

    @pl.when(g == 0)
    def _():
        sums_ref[...] = jnp.zeros((128, 128), jnp.float32)
        cnt_ref[...] = jnp.zeros((1, 128), jnp.float32)

    z = aggp_ref[0] + aggp_ref[1] + y_ref[...]
    dT = jnp.transpose(dinv_ref[...])
    bT = jnp.transpose(batch_ref[...])
    iota_l = lax.broadcasted_iota(jnp.int32, (128, 128), 1)
    bg = bg_ref[...]
    sums = sums_ref[...]
    cnt = cnt_ref[...]
    for i in range(8):
        zi = z[i * 128:(i + 1) * 128, :]
        h = jnp.maximum(zi * dT[:, i:i + 1] + bg, 0.0)
        pt = (bT[:, i:i + 1] == iota_l).astype(jnp.float32)
        sums = sums + lax.dot_general(
            pt, h, (((0,), (0,)), ((), ())),
            preferred_element_type=jnp.float32,
            precision=lax.Precision.HIGHEST)
        cnt = cnt + jnp.sum(pt, axis=0, keepdims=True)
    sums_ref[...] = sums
    cnt_ref[...] = cnt

    @pl.when(g == pl.num_programs(0) - 1)
    def _():
        cntT = jnp.transpose(jnp.maximum(cnt_ref[...], 1.0))
        gf = sums_ref[...] / cntT
        hid = jnp.maximum(
            jnp.dot(gf, W1_ref[...], preferred_element_type=jnp.float32)
            + b1_ref[...], 0.0)
        full = jnp.dot(hid, w2p_ref[...], preferred_element_type=jnp.float32)
        out_ref[...] = jnp.transpose(full[:, 0:1]) + b2_ref[0, 0]


_pool_call = pl.pallas_call(
    _pool_body,
    grid=(GN,),
    in_specs=[
        pl.BlockSpec((NC, RB, D), lambda g: (0, g, 0)),
        pl.BlockSpec((RB, D), lambda g: (g, 0)),
        pl.BlockSpec((RB // 128, 128), lambda g: (g, 0)),
        pl.BlockSpec((RB // 128, 128), lambda g: (g, 0)),
        pl.BlockSpec((1, D), lambda g: (0, 0)),
        pl.BlockSpec((D, D), lambda g: (0, 0)),
        pl.BlockSpec((1, D), lambda g: (0, 0)),
        pl.BlockSpec((D, D), lambda g: (0, 0)),
        pl.BlockSpec((1, 1), lambda g: (0, 0)),
    ],
    out_specs=pl.BlockSpec((1, 128), lambda g: (0, 0)),
    out_shape=jax.ShapeDtypeStruct((1, 128), jnp.float32),
    scratch_shapes=[
        pltpu.VMEM((128, 128), jnp.float32),
        pltpu.VMEM((1, 128), jnp.float32),
    ],
    compiler_params=pltpu.CompilerParams(
        dimension_semantics=("arbitrary",)),
)


def kernel(x, edge_index, batch_vec, W_gcn, b_gcn, W1, b1, W2, b2):
    pad_e = E_PAD - E
    pad_idx = jnp.full((pad_e,), PAD_ROW, jnp.int32)
    src3 = jnp.concatenate([edge_index[0], pad_idx]).reshape(NW, CPT, CH)
    dst3 = jnp.concatenate([edge_index[1], pad_idx]).reshape(NW, CPT, CH)

    degp = _deg_call(dst3.reshape(NW, N_PAD // 16, 16))
    degp_r = degp.reshape(NW, N_PAD // 128, 128)

    x_pad = jnp.concatenate([x, jnp.zeros((N_PAD - N, D), x.dtype)])
    q, dinv2 = _scale_call(x_pad, degp_r, W_gcn)

    aggp = _agg_call(q, src3, dst3)

    batchp = jnp.concatenate(
        [batch_vec, jnp.full((N_PAD - N,), B, jnp.int32)]
    ).reshape(N_PAD // 128, 128)

    w2p = jnp.pad(W2, ((0, 0), (0, D - 1)))
    out = _pool_call(aggp, q, dinv2, batchp,
                     b_gcn.reshape(1, D), W1, b1.reshape(1, D),
                     w2p, b2.reshape(1, 1))
    return out[0, :B]
